# baseline (speedup 1.0000x reference)
import jax
import jax.numpy as jnp
from jax.experimental import pallas as pl
from jax.experimental.pallas import tpu as pltpu

_CHUNK = 2000
_NBUF = 4


def _linear_stream(x_hbm, w_ref, b_ref, o_hbm, xbuf, obuf, insem, outsem):
    n = x_hbm.shape[0]
    nchunks = n // _CHUNK

    def in_copy(i):
        return pltpu.make_async_copy(
            x_hbm.at[pl.ds(i * _CHUNK, _CHUNK)],
            xbuf.at[i % _NBUF],
            insem.at[i % _NBUF],
        )

    def out_copy(i):
        return pltpu.make_async_copy(
            obuf.at[i % _NBUF],
            o_hbm.at[pl.ds(i * _CHUNK, _CHUNK)],
            outsem.at[i % _NBUF],
        )

    for i in range(min(_NBUF, nchunks)):
        in_copy(i).start()

    for i in range(nchunks):
        in_copy(i).wait()
        if i >= _NBUF:
            out_copy(i - _NBUF).wait()
        acc = jax.lax.dot_general(
            xbuf[i % _NBUF],
            w_ref[...],
            (((1,), (0,)), ((), ())),
            precision=jax.lax.Precision.DEFAULT,
            preferred_element_type=jnp.float32,
        )
        obuf[i % _NBUF] = acc + b_ref[...]
        out_copy(i).start()
        if i + _NBUF < nchunks:
            in_copy(i + _NBUF).start()

    for i in range(max(nchunks - _NBUF, 0), nchunks):
        out_copy(i).wait()


def kernel(x, d, W, b):
    del d
    n, d_in = x.shape
    dim = W.shape[1]
    assert n % _CHUNK == 0
    return pl.pallas_call(
        _linear_stream,
        in_specs=[
            pl.BlockSpec(memory_space=pl.ANY),
            pl.BlockSpec((d_in, dim), lambda: (0, 0)),
            pl.BlockSpec((dim,), lambda: (0,)),
        ],
        out_specs=pl.BlockSpec(memory_space=pl.ANY),
        out_shape=jax.ShapeDtypeStruct((n, dim), jnp.float32),
        scratch_shapes=[
            pltpu.VMEM((_NBUF, _CHUNK, d_in), jnp.float32),
            pltpu.VMEM((_NBUF, _CHUNK, dim), jnp.float32),
            pltpu.SemaphoreType.DMA((_NBUF,)),
            pltpu.SemaphoreType.DMA((_NBUF,)),
        ],
    )(x, W, b)


# speedup vs baseline: 1.0226x; 1.0226x over previous
import jax
import jax.numpy as jnp
from jax.experimental import pallas as pl
from jax.experimental.pallas import tpu as pltpu

_CHUNK = 5000
_NBUF = 3


def _linear_stream(x_hbm, w_ref, b_ref, o_hbm, xbuf, obuf, insem, outsem):
    n = x_hbm.shape[0]
    nchunks = n // _CHUNK

    def in_copy(i):
        return pltpu.make_async_copy(
            x_hbm.at[pl.ds(i * _CHUNK, _CHUNK)],
            xbuf.at[i % _NBUF],
            insem.at[i % _NBUF],
        )

    def out_copy(i):
        return pltpu.make_async_copy(
            obuf.at[i % _NBUF],
            o_hbm.at[pl.ds(i * _CHUNK, _CHUNK)],
            outsem.at[i % _NBUF],
        )

    for i in range(min(_NBUF, nchunks)):
        in_copy(i).start()

    for i in range(nchunks):
        in_copy(i).wait()
        if i >= _NBUF:
            out_copy(i - _NBUF).wait()
        acc = jax.lax.dot_general(
            xbuf[i % _NBUF],
            w_ref[...],
            (((1,), (0,)), ((), ())),
            precision=jax.lax.Precision.DEFAULT,
            preferred_element_type=jnp.float32,
        )
        obuf[i % _NBUF] = acc + b_ref[...]
        out_copy(i).start()
        if i + _NBUF < nchunks:
            in_copy(i + _NBUF).start()

    for i in range(max(nchunks - _NBUF, 0), nchunks):
        out_copy(i).wait()


def kernel(x, d, W, b):
    del d
    n, d_in = x.shape
    dim = W.shape[1]
    assert n % _CHUNK == 0
    return pl.pallas_call(
        _linear_stream,
        in_specs=[
            pl.BlockSpec(memory_space=pl.ANY),
            pl.BlockSpec((d_in, dim), lambda: (0, 0)),
            pl.BlockSpec((dim,), lambda: (0,)),
        ],
        out_specs=pl.BlockSpec(memory_space=pl.ANY),
        out_shape=jax.ShapeDtypeStruct((n, dim), jnp.float32),
        scratch_shapes=[
            pltpu.VMEM((_NBUF, _CHUNK, d_in), jnp.float32),
            pltpu.VMEM((_NBUF, _CHUNK, dim), jnp.float32),
            pltpu.SemaphoreType.DMA((_NBUF,)),
            pltpu.SemaphoreType.DMA((_NBUF,)),
        ],
    )(x, W, b)
